# baseline (speedup 1.0000x reference)
import functools

import jax
import jax.numpy as jnp
from jax import lax
from jax.experimental import pallas as pl
from jax.experimental.pallas import tpu as pltpu
from jax.experimental.pallas import tpu_sc as plsc

BATCH = 16384
EMBED = 32
REGS = 0.01
PACK = 128 // EMBED
NROWS = 1000000

NC = 2
NS = 16
LANES = 16
NW = NC * NS
BPW = BATCH // NW
NCHUNK = 2
CROWS = BPW // NCHUNK
CGROUPS = CROWS // LANES

_SC_MESH = plsc.VectorSubcoreMesh(core_axis_name="c", subcore_axis_name="s")


@functools.partial(
    pl.kernel,
    out_type=[
        jax.ShapeDtypeStruct((BATCH,), jnp.float32),
        jax.ShapeDtypeStruct((BATCH,), jnp.float32),
        jax.ShapeDtypeStruct((NW, LANES), jnp.float32),
    ],
    mesh=_SC_MESH,
    compiler_params=pltpu.CompilerParams(needs_layout_passes=False),
    scratch_types=[
        pltpu.VMEM((BPW,), jnp.int32),
        pltpu.VMEM((BPW,), jnp.int32),
        pltpu.VMEM((BPW,), jnp.int32),
        pltpu.VMEM((BPW,), jnp.int32),
        pltpu.VMEM((BPW,), jnp.int32),
        pltpu.VMEM((BPW,), jnp.int32),
        pltpu.VMEM((CROWS, 128), jnp.float32),
        pltpu.VMEM((CROWS, 128), jnp.float32),
        pltpu.VMEM((CROWS, 128), jnp.float32),
        pltpu.VMEM((BPW,), jnp.float32),
        pltpu.VMEM((BPW,), jnp.float32),
        pltpu.VMEM((BPW,), jnp.float32),
        pltpu.VMEM((BPW,), jnp.float32),
        pltpu.VMEM((LANES,), jnp.float32),
        pltpu.SemaphoreType.DMA,
    ],
)
def _sc_lookup(user_hbm, pos_hbm, neg_hbm, uemb_hbm, iemb_hbm, bias_hbm,
               plog_hbm, nlog_hbm, reg_hbm,
               uidx, pidx, nidx, uq, pq, nq, urows, prows, nrows,
               pbias, nbias, plog_v, nlog_v, reg_v, sem):
  wid = lax.axis_index("s") * NC + lax.axis_index("c")
  base = wid * BPW

  pltpu.sync_copy(user_hbm.at[pl.ds(base, BPW)], uidx)
  pltpu.sync_copy(pos_hbm.at[pl.ds(base, BPW)], pidx)
  pltpu.sync_copy(neg_hbm.at[pl.ds(base, BPW)], nidx)

  def quarter_body(g, _):
    sl = pl.ds(g * LANES, LANES)
    uq[sl] = jnp.right_shift(uidx[sl], PACK // 2)
    pq[sl] = jnp.right_shift(pidx[sl], PACK // 2)
    nq[sl] = jnp.right_shift(nidx[sl], PACK // 2)
    return 0

  lax.fori_loop(0, BPW // LANES, quarter_body, 0)

  bias_copies = [
      pltpu.async_copy(bias_hbm.at[pidx], pbias, sem),
      pltpu.async_copy(bias_hbm.at[nidx], nbias, sem),
  ]

  iota = lax.iota(jnp.int32, LANES)
  three = jnp.full((LANES,), PACK - 1, jnp.int32)

  reg_v[...] = jnp.zeros((LANES,), jnp.float32)

  for c in range(NCHUNK):
    csl = pl.ds(c * CROWS, CROWS)
    row_copies = [
        pltpu.async_copy(uemb_hbm.at[uq.at[csl]], urows, sem),
        pltpu.async_copy(iemb_hbm.at[pq.at[csl]], prows, sem),
        pltpu.async_copy(iemb_hbm.at[nq.at[csl]], nrows, sem),
    ]
    for cp in row_copies:
      cp.wait()

    def group_body(g, acc_reg):
      glob = c * CGROUPS + g
      gsl = pl.ds(glob * LANES, LANES)
      rows = g * LANES + iota
      ui = uidx[gsl]
      pi = pidx[gsl]
      ni = nidx[gsl]
      subu = (ui & three) * EMBED
      subp = (pi & three) * EMBED
      subn = (ni & three) * EMBED
      accp = jnp.zeros((LANES,), jnp.float32)
      accn = jnp.zeros((LANES,), jnp.float32)
      accr = acc_reg
      for d in range(EMBED):
        ud = plsc.load_gather(urows, [rows, subu + d])
        pd = plsc.load_gather(prows, [rows, subp + d])
        nd = plsc.load_gather(nrows, [rows, subn + d])
        accp = accp + ud * pd
        accn = accn + ud * nd
        accr = accr + (ud * ud + ud * ud + pd * pd + nd * nd)
      plog_v[gsl] = accp
      nlog_v[gsl] = accn
      return accr

    acc_reg = lax.fori_loop(0, CGROUPS, group_body,
                            jnp.zeros((LANES,), jnp.float32))
    reg_v[...] = reg_v[...] + acc_reg

  for cp in bias_copies:
    cp.wait()

  def bias_body(g, _):
    sl = pl.ds(g * LANES, LANES)
    plog_v[sl] = plog_v[sl] + pbias[sl]
    nlog_v[sl] = nlog_v[sl] + nbias[sl]
    return 0

  lax.fori_loop(0, BPW // LANES, bias_body, 0)

  pltpu.sync_copy(plog_v, plog_hbm.at[pl.ds(base, BPW)])
  pltpu.sync_copy(nlog_v, nlog_hbm.at[pl.ds(base, BPW)])
  pltpu.sync_copy(reg_v, reg_hbm.at[wid])


def _loss_body(plog_ref, nlog_ref, reg_ref, cls_ref, reg_out_ref):
  pos_l = plog_ref[...]
  neg_l = nlog_ref[...]
  pos_bce = (jnp.maximum(pos_l, 0.0) - pos_l
             + jnp.log1p(jnp.exp(-jnp.abs(pos_l))))
  neg_bce = jnp.maximum(neg_l, 0.0) + jnp.log1p(jnp.exp(-jnp.abs(neg_l)))
  cls_ref[...] = (jnp.mean(pos_bce) + jnp.mean(neg_bce)).reshape(1, 1)
  reg_out_ref[...] = ((REGS * 0.5) * jnp.sum(reg_ref[...])).reshape(1, 1)


def kernel(user, pos, neg, user_embedding, item_embedding, bias):
  user = user.astype(jnp.int32)
  pos = pos.astype(jnp.int32)
  neg = neg.astype(jnp.int32)
  uemb = user_embedding.reshape(NROWS // PACK, 128)
  iemb = item_embedding.reshape(NROWS // PACK, 128)
  plog, nlog, regs = _sc_lookup(user, pos, neg, uemb, iemb, bias)
  cls, reg = pl.pallas_call(
      _loss_body,
      out_shape=[jax.ShapeDtypeStruct((1, 1), jnp.float32),
                 jax.ShapeDtypeStruct((1, 1), jnp.float32)],
  )(plog.reshape(128, 128), nlog.reshape(128, 128), regs.reshape(4, 128))
  return (cls[0, 0], reg[0, 0])


# speedup vs baseline: 1.3914x; 1.3914x over previous
import functools

import jax
import jax.numpy as jnp
from jax import lax
from jax.experimental import pallas as pl
from jax.experimental.pallas import tpu as pltpu
from jax.experimental.pallas import tpu_sc as plsc

BATCH = 16384
EMBED = 32
REGS = 0.01
NROWS = 1000000

NC = 2
NS = 16
LANES = 16
NW = NC * NS
BPW = BATCH // NW
NCHUNK = 2
CROWS = BPW // NCHUNK
CGROUPS = CROWS // LANES

_SC_MESH = plsc.VectorSubcoreMesh(core_axis_name="c", subcore_axis_name="s")
_SC_PARAMS = pltpu.CompilerParams(needs_layout_passes=False)


@functools.partial(
    pl.kernel,
    out_type=[
        jax.ShapeDtypeStruct((BATCH,), jnp.float32),
        jax.ShapeDtypeStruct((BATCH,), jnp.float32),
        jax.ShapeDtypeStruct((NW, LANES), jnp.float32),
    ],
    mesh=_SC_MESH,
    compiler_params=_SC_PARAMS,
    scratch_types=[
        pltpu.VMEM((BPW,), jnp.int32),
        pltpu.VMEM((BPW,), jnp.int32),
        pltpu.VMEM((BPW,), jnp.int32),
        pltpu.VMEM((CROWS, 128), jnp.float32),
        pltpu.VMEM((CROWS, 128), jnp.float32),
        pltpu.VMEM((CROWS, 128), jnp.float32),
        pltpu.VMEM((BPW,), jnp.float32),
        pltpu.VMEM((BPW,), jnp.float32),
        pltpu.VMEM((BPW,), jnp.float32),
        pltpu.VMEM((BPW,), jnp.float32),
        pltpu.VMEM((LANES,), jnp.float32),
        pltpu.SemaphoreType.DMA,
    ],
)
def _sc_lookup(user_hbm, pos_hbm, neg_hbm, uemb_hbm, iemb_hbm, bias_hbm,
               plog_hbm, nlog_hbm, reg_hbm,
               uidx, pidx, nidx, urows, prows, nrows,
               pbias, nbias, plog_v, nlog_v, reg_v, sem):
  wid = lax.axis_index("s") * NC + lax.axis_index("c")
  base = wid * BPW

  pltpu.sync_copy(user_hbm.at[pl.ds(base, BPW)], uidx)
  pltpu.sync_copy(pos_hbm.at[pl.ds(base, BPW)], pidx)
  pltpu.sync_copy(neg_hbm.at[pl.ds(base, BPW)], nidx)

  bias_copies = [
      pltpu.async_copy(bias_hbm.at[pidx], pbias, sem),
      pltpu.async_copy(bias_hbm.at[nidx], nbias, sem),
  ]

  iota = lax.iota(jnp.int32, LANES)

  reg_v[...] = jnp.zeros((LANES,), jnp.float32)

  for c in range(NCHUNK):
    csl = pl.ds(c * CROWS, CROWS)
    row_copies = [
        pltpu.async_copy(uemb_hbm.at[uidx.at[csl]], urows, sem),
        pltpu.async_copy(iemb_hbm.at[pidx.at[csl]], prows, sem),
        pltpu.async_copy(iemb_hbm.at[nidx.at[csl]], nrows, sem),
    ]
    for cp in row_copies:
      cp.wait()

    def group_body(g, acc_reg):
      glob = c * CGROUPS + g
      gsl = pl.ds(glob * LANES, LANES)
      rows = g * LANES + iota
      accp = jnp.zeros((LANES,), jnp.float32)
      accn = jnp.zeros((LANES,), jnp.float32)
      accr = acc_reg
      for d in range(EMBED):
        dv = jnp.full((LANES,), d, jnp.int32)
        ud = plsc.load_gather(urows, [rows, dv])
        pd = plsc.load_gather(prows, [rows, dv])
        nd = plsc.load_gather(nrows, [rows, dv])
        accp = accp + ud * pd
        accn = accn + ud * nd
        accr = accr + (ud * ud + ud * ud + pd * pd + nd * nd)
      plog_v[gsl] = accp
      nlog_v[gsl] = accn
      return accr

    acc_reg = lax.fori_loop(0, CGROUPS, group_body,
                            jnp.zeros((LANES,), jnp.float32))
    reg_v[...] = reg_v[...] + acc_reg

  for cp in bias_copies:
    cp.wait()

  def bias_body(g, _):
    sl = pl.ds(g * LANES, LANES)
    plog_v[sl] = plog_v[sl] + pbias[sl]
    nlog_v[sl] = nlog_v[sl] + nbias[sl]
    return 0

  lax.fori_loop(0, BPW // LANES, bias_body, 0)

  pltpu.sync_copy(plog_v, plog_hbm.at[pl.ds(base, BPW)])
  pltpu.sync_copy(nlog_v, nlog_hbm.at[pl.ds(base, BPW)])
  pltpu.sync_copy(reg_v, reg_hbm.at[wid])


def _loss_body(plog_ref, nlog_ref, reg_ref, cls_ref, reg_out_ref):
  pos_l = plog_ref[...]
  neg_l = nlog_ref[...]
  pos_bce = (jnp.maximum(pos_l, 0.0) - pos_l
             + jnp.log1p(jnp.exp(-jnp.abs(pos_l))))
  neg_bce = jnp.maximum(neg_l, 0.0) + jnp.log1p(jnp.exp(-jnp.abs(neg_l)))
  cls_ref[...] = (jnp.mean(pos_bce) + jnp.mean(neg_bce)).reshape(1, 1)
  reg_out_ref[...] = ((REGS * 0.5) * jnp.sum(reg_ref[...])).reshape(1, 1)


TC_W = 2048
TC_GRID = -(-NROWS // TC_W)


def _repack_body(ut_ref, it_ref, su_ref, si_ref):
  su_ref[:, 0:EMBED] = jnp.transpose(ut_ref[...])
  si_ref[:, 0:EMBED] = jnp.transpose(it_ref[...])


_tc_repack = pl.pallas_call(
    _repack_body,
    grid=(TC_GRID,),
    in_specs=[
        pl.BlockSpec((EMBED, TC_W), lambda i: (0, i)),
        pl.BlockSpec((EMBED, TC_W), lambda i: (0, i)),
    ],
    out_specs=[
        pl.BlockSpec((TC_W, 128), lambda i: (i, 0)),
        pl.BlockSpec((TC_W, 128), lambda i: (i, 0)),
    ],
    out_shape=[
        jax.ShapeDtypeStruct((NROWS, 128), jnp.float32),
        jax.ShapeDtypeStruct((NROWS, 128), jnp.float32),
    ],
)


def kernel(user, pos, neg, user_embedding, item_embedding, bias):
  user = user.astype(jnp.int32)
  pos = pos.astype(jnp.int32)
  neg = neg.astype(jnp.int32)
  su, si = _tc_repack(user_embedding.T, item_embedding.T)
  plog, nlog, regs = _sc_lookup(user, pos, neg, su, si, bias)
  cls, reg = pl.pallas_call(
      _loss_body,
      out_shape=[jax.ShapeDtypeStruct((1, 1), jnp.float32),
                 jax.ShapeDtypeStruct((1, 1), jnp.float32)],
  )(plog.reshape(128, 128), nlog.reshape(128, 128), regs.reshape(4, 128))
  return (cls[0, 0], reg[0, 0])


# speedup vs baseline: 2.0082x; 1.4433x over previous
import functools

import jax
import jax.numpy as jnp
from jax import lax
from jax.experimental import pallas as pl
from jax.experimental.pallas import tpu as pltpu
from jax.experimental.pallas import tpu_sc as plsc

BATCH = 16384
EMBED = 32
REGS = 0.01
NROWS = 1000000

NC = 2
NS = 16
LANES = 16
NW = NC * NS
BPW = BATCH // NW
NCHUNK = 2
CROWS = BPW // NCHUNK
CGROUPS = CROWS // LANES

_SC_MESH = plsc.VectorSubcoreMesh(core_axis_name="c", subcore_axis_name="s")
_SC_PARAMS = pltpu.CompilerParams(needs_layout_passes=False)


@functools.partial(
    pl.kernel,
    out_type=[
        jax.ShapeDtypeStruct((BATCH,), jnp.float32),
        jax.ShapeDtypeStruct((BATCH,), jnp.float32),
        jax.ShapeDtypeStruct((NW, LANES), jnp.float32),
    ],
    mesh=_SC_MESH,
    compiler_params=_SC_PARAMS,
    scratch_types=[
        pltpu.VMEM((BPW,), jnp.int32),
        pltpu.VMEM((BPW,), jnp.int32),
        pltpu.VMEM((BPW,), jnp.int32),
        pltpu.VMEM((CROWS, 128), jnp.float32),
        pltpu.VMEM((CROWS, 128), jnp.float32),
        pltpu.VMEM((CROWS, 128), jnp.float32),
        pltpu.VMEM((BPW,), jnp.float32),
        pltpu.VMEM((BPW,), jnp.float32),
        pltpu.VMEM((BPW,), jnp.float32),
        pltpu.VMEM((BPW,), jnp.float32),
        pltpu.VMEM((LANES,), jnp.float32),
        pltpu.SemaphoreType.DMA,
    ],
)
def _sc_lookup(user_hbm, pos_hbm, neg_hbm, uemb_hbm, iemb_hbm, bias_hbm,
               plog_hbm, nlog_hbm, reg_hbm,
               uidx, pidx, nidx, urows, prows, nrows,
               pbias, nbias, plog_v, nlog_v, reg_v, sem):
  wid = lax.axis_index("s") * NC + lax.axis_index("c")
  base = wid * BPW

  pltpu.sync_copy(user_hbm.at[pl.ds(base, BPW)], uidx)
  pltpu.sync_copy(pos_hbm.at[pl.ds(base, BPW)], pidx)
  pltpu.sync_copy(neg_hbm.at[pl.ds(base, BPW)], nidx)

  bias_copies = [
      pltpu.async_copy(bias_hbm.at[pidx], pbias, sem),
      pltpu.async_copy(bias_hbm.at[nidx], nbias, sem),
  ]

  iota = lax.iota(jnp.int32, LANES)

  reg_v[...] = jnp.zeros((LANES,), jnp.float32)

  for c in range(NCHUNK):
    csl = pl.ds(c * CROWS, CROWS)
    row_copies = [
        pltpu.async_copy(uemb_hbm.at[uidx.at[csl]], urows, sem),
        pltpu.async_copy(iemb_hbm.at[pidx.at[csl]], prows, sem),
        pltpu.async_copy(iemb_hbm.at[nidx.at[csl]], nrows, sem),
    ]
    for cp in row_copies:
      cp.wait()

    def group_body(g, acc_reg):
      glob = c * CGROUPS + g
      gsl = pl.ds(glob * LANES, LANES)
      rows = g * LANES + iota
      accp = jnp.zeros((LANES,), jnp.float32)
      accn = jnp.zeros((LANES,), jnp.float32)
      accr = acc_reg
      for d in range(EMBED):
        dv = jnp.full((LANES,), d, jnp.int32)
        ud = plsc.load_gather(urows, [rows, dv])
        pd = plsc.load_gather(prows, [rows, dv])
        nd = plsc.load_gather(nrows, [rows, dv])
        accp = accp + ud * pd
        accn = accn + ud * nd
        accr = accr + (ud * ud + ud * ud + pd * pd + nd * nd)
      plog_v[gsl] = accp
      nlog_v[gsl] = accn
      return accr

    acc_reg = lax.fori_loop(0, CGROUPS, group_body,
                            jnp.zeros((LANES,), jnp.float32))
    reg_v[...] = reg_v[...] + acc_reg

  for cp in bias_copies:
    cp.wait()

  def bias_body(g, _):
    sl = pl.ds(g * LANES, LANES)
    plog_v[sl] = plog_v[sl] + pbias[sl]
    nlog_v[sl] = nlog_v[sl] + nbias[sl]
    return 0

  lax.fori_loop(0, BPW // LANES, bias_body, 0)

  pltpu.sync_copy(plog_v, plog_hbm.at[pl.ds(base, BPW)])
  pltpu.sync_copy(nlog_v, nlog_hbm.at[pl.ds(base, BPW)])
  pltpu.sync_copy(reg_v, reg_hbm.at[wid])


def _loss_body(plog_ref, nlog_ref, reg_ref, cls_ref, reg_out_ref):
  pos_l = plog_ref[...]
  neg_l = nlog_ref[...]
  pos_bce = (jnp.maximum(pos_l, 0.0) - pos_l
             + jnp.log1p(jnp.exp(-jnp.abs(pos_l))))
  neg_bce = jnp.maximum(neg_l, 0.0) + jnp.log1p(jnp.exp(-jnp.abs(neg_l)))
  cls_ref[...] = (jnp.mean(pos_bce) + jnp.mean(neg_bce)).reshape(1, 1)
  reg_out_ref[...] = ((REGS * 0.5) * jnp.sum(reg_ref[...])).reshape(1, 1)


TC_W = 8192
TC_GRID = -(-NROWS // TC_W)


def _repack_body(ut_ref, it_ref, su_ref, si_ref):
  su_ref[:, 0:EMBED] = jnp.transpose(ut_ref[...])
  si_ref[:, 0:EMBED] = jnp.transpose(it_ref[...])


_tc_repack = pl.pallas_call(
    _repack_body,
    grid=(TC_GRID,),
    in_specs=[
        pl.BlockSpec((EMBED, TC_W), lambda i: (0, i)),
        pl.BlockSpec((EMBED, TC_W), lambda i: (0, i)),
    ],
    out_specs=[
        pl.BlockSpec((TC_W, 128), lambda i: (i, 0)),
        pl.BlockSpec((TC_W, 128), lambda i: (i, 0)),
    ],
    out_shape=[
        jax.ShapeDtypeStruct((NROWS, 128), jnp.float32),
        jax.ShapeDtypeStruct((NROWS, 128), jnp.float32),
    ],
)


def kernel(user, pos, neg, user_embedding, item_embedding, bias):
  user = user.astype(jnp.int32)
  pos = pos.astype(jnp.int32)
  neg = neg.astype(jnp.int32)
  su, si = _tc_repack(user_embedding.T, item_embedding.T)
  plog, nlog, regs = _sc_lookup(user, pos, neg, su, si, bias)
  cls, reg = pl.pallas_call(
      _loss_body,
      out_shape=[jax.ShapeDtypeStruct((1, 1), jnp.float32),
                 jax.ShapeDtypeStruct((1, 1), jnp.float32)],
  )(plog.reshape(128, 128), nlog.reshape(128, 128), regs.reshape(4, 128))
  return (cls[0, 0], reg[0, 0])


# speedup vs baseline: 2.0688x; 1.0302x over previous
import functools

import jax
import jax.numpy as jnp
from jax import lax
from jax.experimental import pallas as pl
from jax.experimental.pallas import tpu as pltpu
from jax.experimental.pallas import tpu_sc as plsc

BATCH = 16384
EMBED = 32
REGS = 0.01
NROWS = 1000000

NC = 2
NS = 16
LANES = 16
NW = NC * NS
BPW = BATCH // NW
NCHUNK = 2
CROWS = BPW // NCHUNK
CGROUPS = CROWS // LANES

_SC_MESH = plsc.VectorSubcoreMesh(core_axis_name="c", subcore_axis_name="s")
_SC_PARAMS = pltpu.CompilerParams(needs_layout_passes=False)


@functools.partial(
    pl.kernel,
    out_type=[
        jax.ShapeDtypeStruct((BATCH,), jnp.float32),
        jax.ShapeDtypeStruct((BATCH,), jnp.float32),
        jax.ShapeDtypeStruct((NW, LANES), jnp.float32),
    ],
    mesh=_SC_MESH,
    compiler_params=_SC_PARAMS,
    scratch_types=[
        pltpu.VMEM((BPW,), jnp.int32),
        pltpu.VMEM((BPW,), jnp.int32),
        pltpu.VMEM((BPW,), jnp.int32),
        pltpu.VMEM((CROWS, 128), jnp.float32),
        pltpu.VMEM((CROWS, 128), jnp.float32),
        pltpu.VMEM((CROWS, 128), jnp.float32),
        pltpu.VMEM((BPW,), jnp.float32),
        pltpu.VMEM((BPW,), jnp.float32),
        pltpu.VMEM((BPW,), jnp.float32),
        pltpu.VMEM((BPW,), jnp.float32),
        pltpu.VMEM((LANES,), jnp.float32),
        pltpu.SemaphoreType.DMA,
    ],
)
def _sc_lookup(user_hbm, pos_hbm, neg_hbm, uemb_hbm, iemb_hbm, bias_hbm,
               plog_hbm, nlog_hbm, reg_hbm,
               uidx, pidx, nidx, urows, prows, nrows,
               pbias, nbias, plog_v, nlog_v, reg_v, sem):
  wid = lax.axis_index("s") * NC + lax.axis_index("c")
  base = wid * BPW

  pltpu.sync_copy(user_hbm.at[pl.ds(base, BPW)], uidx)
  pltpu.sync_copy(pos_hbm.at[pl.ds(base, BPW)], pidx)
  pltpu.sync_copy(neg_hbm.at[pl.ds(base, BPW)], nidx)

  bias_copies = [
      pltpu.async_copy(bias_hbm.at[pidx], pbias, sem),
      pltpu.async_copy(bias_hbm.at[nidx], nbias, sem),
  ]

  iota = lax.iota(jnp.int32, LANES)

  reg_v[...] = jnp.zeros((LANES,), jnp.float32)

  for c in range(NCHUNK):
    csl = pl.ds(c * CROWS, CROWS)
    row_copies = [
        pltpu.async_copy(uemb_hbm.at[uidx.at[csl]], urows, sem),
        pltpu.async_copy(iemb_hbm.at[pidx.at[csl]], prows, sem),
        pltpu.async_copy(iemb_hbm.at[nidx.at[csl]], nrows, sem),
    ]
    for cp in row_copies:
      cp.wait()

    def group_body(g, acc_reg):
      glob = c * CGROUPS + g
      gsl = pl.ds(glob * LANES, LANES)
      rows = g * LANES + iota
      accp = jnp.zeros((LANES,), jnp.float32)
      accn = jnp.zeros((LANES,), jnp.float32)
      accr = acc_reg
      for d in range(EMBED):
        dv = jnp.full((LANES,), d, jnp.int32)
        ud = plsc.load_gather(urows, [rows, dv])
        pd = plsc.load_gather(prows, [rows, dv])
        nd = plsc.load_gather(nrows, [rows, dv])
        accp = accp + ud * pd
        accn = accn + ud * nd
        accr = accr + (ud * ud + ud * ud + pd * pd + nd * nd)
      plog_v[gsl] = accp
      nlog_v[gsl] = accn
      return accr

    acc_reg = lax.fori_loop(0, CGROUPS, group_body,
                            jnp.zeros((LANES,), jnp.float32))
    reg_v[...] = reg_v[...] + acc_reg

  for cp in bias_copies:
    cp.wait()

  def bias_body(g, _):
    sl = pl.ds(g * LANES, LANES)
    plog_v[sl] = plog_v[sl] + pbias[sl]
    nlog_v[sl] = nlog_v[sl] + nbias[sl]
    return 0

  lax.fori_loop(0, BPW // LANES, bias_body, 0)

  pltpu.sync_copy(plog_v, plog_hbm.at[pl.ds(base, BPW)])
  pltpu.sync_copy(nlog_v, nlog_hbm.at[pl.ds(base, BPW)])
  pltpu.sync_copy(reg_v, reg_hbm.at[wid])


def _loss_body(plog_ref, nlog_ref, reg_ref, cls_ref, reg_out_ref):
  pos_l = plog_ref[...]
  neg_l = nlog_ref[...]
  pos_bce = (jnp.maximum(pos_l, 0.0) - pos_l
             + jnp.log1p(jnp.exp(-jnp.abs(pos_l))))
  neg_bce = jnp.maximum(neg_l, 0.0) + jnp.log1p(jnp.exp(-jnp.abs(neg_l)))
  cls_ref[...] = (jnp.mean(pos_bce) + jnp.mean(neg_bce)).reshape(1, 1)
  reg_out_ref[...] = ((REGS * 0.5) * jnp.sum(reg_ref[...])).reshape(1, 1)


TC_W = 16384
TC_GRID = -(-NROWS // TC_W)


def _repack_body(ut_ref, it_ref, su_ref, si_ref):
  su_ref[:, 0:EMBED] = jnp.transpose(ut_ref[...])
  si_ref[:, 0:EMBED] = jnp.transpose(it_ref[...])


_tc_repack = pl.pallas_call(
    _repack_body,
    grid=(TC_GRID,),
    in_specs=[
        pl.BlockSpec((EMBED, TC_W), lambda i: (0, i)),
        pl.BlockSpec((EMBED, TC_W), lambda i: (0, i)),
    ],
    out_specs=[
        pl.BlockSpec((TC_W, 128), lambda i: (i, 0)),
        pl.BlockSpec((TC_W, 128), lambda i: (i, 0)),
    ],
    out_shape=[
        jax.ShapeDtypeStruct((NROWS, 128), jnp.float32),
        jax.ShapeDtypeStruct((NROWS, 128), jnp.float32),
    ],
)


def kernel(user, pos, neg, user_embedding, item_embedding, bias):
  user = user.astype(jnp.int32)
  pos = pos.astype(jnp.int32)
  neg = neg.astype(jnp.int32)
  su, si = _tc_repack(user_embedding.T, item_embedding.T)
  plog, nlog, regs = _sc_lookup(user, pos, neg, su, si, bias)
  cls, reg = pl.pallas_call(
      _loss_body,
      out_shape=[jax.ShapeDtypeStruct((1, 1), jnp.float32),
                 jax.ShapeDtypeStruct((1, 1), jnp.float32)],
  )(plog.reshape(128, 128), nlog.reshape(128, 128), regs.reshape(4, 128))
  return (cls[0, 0], reg[0, 0])
